# baseline (speedup 1.0000x reference)
import jax
import jax.numpy as jnp
from jax import lax
from jax.experimental import pallas as pl
from jax.experimental.pallas import tpu as pltpu
from jax.experimental.pallas import tpu_sc as plsc

N = 10000
D = 128
DH = D // 2
E = 320000
NC = 2
NS = 16
B = 128
NBT = E // B
BPT = NBT // NS
XTRA = NBT - BPT * NS
MAXB = BPT + 1
NPAD = 10240
RPT = NPAD // NS
RB = 1024
NB = 6

_UNTILED = pltpu.CompilerParams(use_tc_tiling_on_sc=False)


def _tile_range(sid):
    start = BPT * sid + jnp.minimum(sid, XTRA)
    nb = BPT + (sid < XTRA).astype(jnp.int32)
    return start, nb


def _deg_body(ei_hbm, d0_hbm, d1_hbm, col_v, ones_v, zero_v, deg_sh, h0, h1):
    cid = lax.axis_index("c")
    sid = lax.axis_index("s")
    start, nb = _tile_range(sid)
    def zb(i, c):
        zero_v[pl.ds(i * 16, 16)] = jnp.zeros((16,), jnp.float32)
        return c
    lax.fori_loop(0, RPT // 16, zb, 0)
    pltpu.sync_copy(zero_v, deg_sh.at[pl.ds(sid * RPT, RPT)])
    def ob(i, c):
        ones_v[pl.ds(i * 16, 16)] = jnp.ones((16,), jnp.float32)
        return c
    lax.fori_loop(0, B // 16, ob, 0)
    plsc.subcore_barrier()
    @pl.when(sid < XTRA)
    def _():
        pltpu.sync_copy(ei_hbm.at[pl.ds(start, MAXB), pl.ds(1, 1)], col_v)
    @pl.when(sid >= XTRA)
    def _():
        pltpu.sync_copy(ei_hbm.at[pl.ds(start, BPT), pl.ds(1, 1)],
                        col_v.at[pl.ds(0, BPT)])
    half = (nb + 1) // 2
    j0 = cid * half
    cnt = jnp.where(cid == 0, half, nb - half)
    hsem = (h0, h1)
    def _wait_one(u):
        pltpu.make_async_copy(ones_v, deg_sh.at[col_v.at[0, 0]],
                              hsem[u]).wait()
    def douter(t, c):
        for u in range(2):
            k = t * 2 + u
            @pl.when(k >= 2)
            def _():
                _wait_one(u)
            pltpu.async_copy(ones_v, deg_sh.at[col_v.at[j0 + k, 0]],
                             hsem[u], add=True)
        return c
    lax.fori_loop(0, cnt // 2, douter, 0)
    _wait_one(0)
    _wait_one(1)
    @pl.when(cnt % 2 == 1)
    def _():
        pltpu.sync_copy(ones_v,
                        deg_sh.at[col_v.at[j0 + (cnt // 2) * 2, 0]], add=True)
    plsc.subcore_barrier()
    my_rows = deg_sh.at[pl.ds(sid * RPT, RPT)]
    @pl.when(cid == 0)
    def _():
        pltpu.sync_copy(my_rows, d0_hbm.at[pl.ds(sid * RPT, RPT)])
    @pl.when(cid == 1)
    def _():
        pltpu.sync_copy(my_rows, d1_hbm.at[pl.ds(sid * RPT, RPT)])


_deg_call = pl.kernel(
    _deg_body,
    out_type=(jax.ShapeDtypeStruct((NPAD,), jnp.float32),
              jax.ShapeDtypeStruct((NPAD,), jnp.float32)),
    mesh=plsc.VectorSubcoreMesh(core_axis_name="c", subcore_axis_name="s",
                                num_cores=NC, num_subcores=NS),
    scratch_types=[
        pltpu.VMEM((MAXB, 1, B), jnp.int32),
        pltpu.VMEM((B,), jnp.float32),
        pltpu.VMEM((RPT,), jnp.float32),
        pltpu.VMEM_SHARED((NPAD,), jnp.float32),
        pltpu.SemaphoreType.DMA,
        pltpu.SemaphoreType.DMA,
    ],
    compiler_params=_UNTILED,
)


def _gs_body(y_hbm, ei_hbm, s_hbm,
             row_v, col_v, b0, b1, b2, b3, b4, b5, s_sh,
             g0, g1, g2, g3, g4, g5):
    cid = lax.axis_index("c")
    sid = lax.axis_index("s")
    bufs = (b0, b1, b2, b3, b4, b5)
    gsem = (g0, g1, g2, g3, g4, g5)
    start, nb = _tile_range(sid)
    def fz(i, c):
        for k in range(DH // 16):
            b0[i, pl.ds(k * 16, 16)] = jnp.zeros((16,), jnp.float32)
        return c
    lax.fori_loop(0, B, fz, 0)
    def zb(t, c):
        pltpu.sync_copy(b0, s_sh.at[pl.ds(sid * RPT + t * B, B)])
        return c
    lax.fori_loop(0, RPT // B, zb, 0)
    plsc.subcore_barrier()
    @pl.when(sid < XTRA)
    def _():
        pltpu.sync_copy(ei_hbm.at[pl.ds(start, MAXB), pl.ds(0, 1)], row_v)
        pltpu.sync_copy(ei_hbm.at[pl.ds(start, MAXB), pl.ds(1, 1)], col_v)
    @pl.when(sid >= XTRA)
    def _():
        pltpu.sync_copy(ei_hbm.at[pl.ds(start, BPT), pl.ds(0, 1)],
                        row_v.at[pl.ds(0, BPT)])
        pltpu.sync_copy(ei_hbm.at[pl.ds(start, BPT), pl.ds(1, 1)],
                        col_v.at[pl.ds(0, BPT)])
    cvec = jnp.full((16,), 0, jnp.int32) + cid
    def rm(j, c):
        for k in range(B // 16):
            v = row_v[j, 0, pl.ds(k * 16, 16)]
            row_v[j, 0, pl.ds(k * 16, 16)] = v + v + cvec
        return c
    lax.fori_loop(0, nb, rm, 0)

    for u in range(NB):
        pltpu.async_copy(y_hbm.at[row_v.at[u, 0]], bufs[u], gsem[u])
    def outer(t, c):
        for u in range(NB):
            j = t * NB + u
            pltpu.make_async_copy(y_hbm.at[row_v.at[j, 0]], bufs[u],
                                  gsem[u]).wait()
            pltpu.sync_copy(bufs[u], s_sh.at[col_v.at[j, 0]], add=True)
            @pl.when(j + NB < nb)
            def _():
                pltpu.async_copy(y_hbm.at[row_v.at[j + NB, 0]], bufs[u],
                                 gsem[u])
        return c
    lax.fori_loop(0, BPT // NB, outer, 0)
    @pl.when(sid < XTRA)
    def _():
        pltpu.make_async_copy(y_hbm.at[row_v.at[BPT, 0]], bufs[0],
                              gsem[0]).wait()
        pltpu.sync_copy(bufs[0], s_sh.at[col_v.at[BPT, 0]], add=True)
    plsc.subcore_barrier()
    pltpu.sync_copy(s_sh.at[pl.ds(sid * RPT, RPT)],
                    s_hbm.at[pl.ds(sid * RPT, RPT), pl.ds(cid * DH, DH)])


_gs_call = pl.kernel(
    _gs_body,
    out_type=jax.ShapeDtypeStruct((NPAD, D), jnp.float32),
    mesh=plsc.VectorSubcoreMesh(core_axis_name="c", subcore_axis_name="s",
                                num_cores=NC, num_subcores=NS),
    scratch_types=(
        [pltpu.VMEM((MAXB, 1, B), jnp.int32)] * 2
        + [pltpu.VMEM((B, DH), jnp.float32)] * NB
        + [pltpu.VMEM_SHARED((NPAD, DH), jnp.float32)]
        + [pltpu.SemaphoreType.DMA] * NB
    ),
    compiler_params=_UNTILED,
)


def _sigmoid(x):
    return 1.0 / (1.0 + jnp.exp(-x))


def _dis_col(dd_ref):
    dis = lax.rsqrt(dd_ref[...] + 1.0)
    return jnp.transpose(dis.reshape(1, RB))


def _y_body(x_ref, w_ref, wihT_ref, bi_ref, bh_ref, dd_ref, y_ref, wev_ref):
    @pl.when(pl.program_id(0) == 0)
    def _():
        g = jnp.dot(w_ref[...], wihT_ref[...],
                    preferred_element_type=jnp.float32)
        g = g + bi_ref[...] + bh_ref[...]
        i = g[:, :D]
        gg = g[:, 2 * D:3 * D]
        o = g[:, 3 * D:]
        c = _sigmoid(i) * jnp.tanh(gg)
        wev_ref[...] = _sigmoid(o) * jnp.tanh(c)
    y_ref[...] = jnp.dot(x_ref[...], wev_ref[...],
                         preferred_element_type=jnp.float32) * _dis_col(dd_ref)


def _y_call(X, W, W_ihT, bi, bh, dd):
    return pl.pallas_call(
        _y_body,
        grid=((N + RB - 1) // RB,),
        in_specs=[
            pl.BlockSpec((RB, D), lambda i: (i, 0)),
            pl.BlockSpec((D, D), lambda i: (0, 0)),
            pl.BlockSpec((D, 4 * D), lambda i: (0, 0)),
            pl.BlockSpec((1, 4 * D), lambda i: (0, 0)),
            pl.BlockSpec((1, 4 * D), lambda i: (0, 0)),
            pl.BlockSpec((RB,), lambda i: (i,)),
        ],
        out_specs=pl.BlockSpec((RB, D), lambda i: (i, 0)),
        out_shape=jax.ShapeDtypeStruct((N, D), jnp.float32),
        scratch_shapes=[pltpu.VMEM((D, D), jnp.float32)],
    )(X, W, W_ihT, bi, bh, dd)


def _out_body(s_ref, y_ref, dd_ref, o_ref):
    o_ref[...] = (s_ref[...] + y_ref[...]) * _dis_col(dd_ref)


def _out_call(s, y, dd):
    return pl.pallas_call(
        _out_body,
        grid=((N + RB - 1) // RB,),
        in_specs=[
            pl.BlockSpec((RB, D), lambda i: (i, 0)),
            pl.BlockSpec((RB, D), lambda i: (i, 0)),
            pl.BlockSpec((RB,), lambda i: (i,)),
        ],
        out_specs=pl.BlockSpec((RB, D), lambda i: (i, 0)),
        out_shape=jax.ShapeDtypeStruct((N, D), jnp.float32),
    )(s, y, dd)


def kernel(X, edge_index, W, W_ih, W_hh, b_ih, b_hh):
    ei3 = edge_index.reshape(2, NBT, B).transpose(1, 0, 2)

    d0, d1 = _deg_call(ei3)
    dd = d0 + d1
    Y = _y_call(X, W, W_ih.T, b_ih.reshape(1, 4 * D), b_hh.reshape(1, 4 * D),
                dd)
    y_view = Y.reshape(2 * N, DH)
    S = _gs_call(y_view, ei3)
    return _out_call(S, Y, dd)


# speedup vs baseline: 1.0026x; 1.0026x over previous
import jax
import jax.numpy as jnp
from jax import lax
from jax.experimental import pallas as pl
from jax.experimental.pallas import tpu as pltpu
from jax.experimental.pallas import tpu_sc as plsc

N = 10000
D = 128
DH = D // 2
E = 320000
NC = 2
NS = 16
B = 128
NBT = E // B
BPT = NBT // NS
XTRA = NBT - BPT * NS
MAXB = BPT + 1
NPAD = 10240
RPT = NPAD // NS
RB = 1024
NB = 4

_UNTILED = pltpu.CompilerParams(use_tc_tiling_on_sc=False)


def _tile_range(sid):
    start = BPT * sid + jnp.minimum(sid, XTRA)
    nb = BPT + (sid < XTRA).astype(jnp.int32)
    return start, nb


def _deg_body(ei_hbm, d0_hbm, d1_hbm, col_v, ones_v, zero_v, deg_sh, h0, h1):
    cid = lax.axis_index("c")
    sid = lax.axis_index("s")
    start, nb = _tile_range(sid)
    def zb(i, c):
        zero_v[pl.ds(i * 16, 16)] = jnp.zeros((16,), jnp.float32)
        return c
    lax.fori_loop(0, RPT // 16, zb, 0)
    pltpu.sync_copy(zero_v, deg_sh.at[pl.ds(sid * RPT, RPT)])
    def ob(i, c):
        ones_v[pl.ds(i * 16, 16)] = jnp.ones((16,), jnp.float32)
        return c
    lax.fori_loop(0, B // 16, ob, 0)
    plsc.subcore_barrier()
    @pl.when(sid < XTRA)
    def _():
        pltpu.sync_copy(ei_hbm.at[pl.ds(start, MAXB), pl.ds(1, 1)], col_v)
    @pl.when(sid >= XTRA)
    def _():
        pltpu.sync_copy(ei_hbm.at[pl.ds(start, BPT), pl.ds(1, 1)],
                        col_v.at[pl.ds(0, BPT)])
    half = (nb + 1) // 2
    j0 = cid * half
    cnt = jnp.where(cid == 0, half, nb - half)
    hsem = (h0, h1)
    def _wait_one(u):
        pltpu.make_async_copy(ones_v, deg_sh.at[col_v.at[0, 0]],
                              hsem[u]).wait()
    def douter(t, c):
        for u in range(2):
            k = t * 2 + u
            @pl.when(k >= 2)
            def _():
                _wait_one(u)
            pltpu.async_copy(ones_v, deg_sh.at[col_v.at[j0 + k, 0]],
                             hsem[u], add=True)
        return c
    lax.fori_loop(0, cnt // 2, douter, 0)
    _wait_one(0)
    _wait_one(1)
    @pl.when(cnt % 2 == 1)
    def _():
        pltpu.sync_copy(ones_v,
                        deg_sh.at[col_v.at[j0 + (cnt // 2) * 2, 0]], add=True)
    plsc.subcore_barrier()
    my_rows = deg_sh.at[pl.ds(sid * RPT, RPT)]
    @pl.when(cid == 0)
    def _():
        pltpu.sync_copy(my_rows, d0_hbm.at[pl.ds(sid * RPT, RPT)])
    @pl.when(cid == 1)
    def _():
        pltpu.sync_copy(my_rows, d1_hbm.at[pl.ds(sid * RPT, RPT)])


_deg_call = pl.kernel(
    _deg_body,
    out_type=(jax.ShapeDtypeStruct((NPAD,), jnp.float32),
              jax.ShapeDtypeStruct((NPAD,), jnp.float32)),
    mesh=plsc.VectorSubcoreMesh(core_axis_name="c", subcore_axis_name="s",
                                num_cores=NC, num_subcores=NS),
    scratch_types=[
        pltpu.VMEM((MAXB, 1, B), jnp.int32),
        pltpu.VMEM((B,), jnp.float32),
        pltpu.VMEM((RPT,), jnp.float32),
        pltpu.VMEM_SHARED((NPAD,), jnp.float32),
        pltpu.SemaphoreType.DMA,
        pltpu.SemaphoreType.DMA,
    ],
    compiler_params=_UNTILED,
)


def _gs_body(y_hbm, ei_hbm, s_hbm,
             row_v, col_v, b0, b1, b2, b3, s_sh,
             g0, g1, g2, g3):
    cid = lax.axis_index("c")
    sid = lax.axis_index("s")
    bufs = (b0, b1, b2, b3)
    gsem = (g0, g1, g2, g3)
    start, nb = _tile_range(sid)
    def fz(i, c):
        for k in range(DH // 16):
            b0[i, pl.ds(k * 16, 16)] = jnp.zeros((16,), jnp.float32)
        return c
    lax.fori_loop(0, B, fz, 0)
    def zb(t, c):
        pltpu.sync_copy(b0, s_sh.at[pl.ds(sid * RPT + t * B, B)])
        return c
    lax.fori_loop(0, RPT // B, zb, 0)
    plsc.subcore_barrier()
    @pl.when(sid < XTRA)
    def _():
        pltpu.sync_copy(ei_hbm.at[pl.ds(start, MAXB), pl.ds(0, 1)], row_v)
        pltpu.sync_copy(ei_hbm.at[pl.ds(start, MAXB), pl.ds(1, 1)], col_v)
    @pl.when(sid >= XTRA)
    def _():
        pltpu.sync_copy(ei_hbm.at[pl.ds(start, BPT), pl.ds(0, 1)],
                        row_v.at[pl.ds(0, BPT)])
        pltpu.sync_copy(ei_hbm.at[pl.ds(start, BPT), pl.ds(1, 1)],
                        col_v.at[pl.ds(0, BPT)])
    cvec = jnp.full((16,), 0, jnp.int32) + cid
    def rm(j, c):
        for k in range(B // 16):
            v = row_v[j, 0, pl.ds(k * 16, 16)]
            row_v[j, 0, pl.ds(k * 16, 16)] = v + v + cvec
        return c
    lax.fori_loop(0, nb, rm, 0)

    for u in range(NB):
        pltpu.async_copy(y_hbm.at[row_v.at[u, 0]], bufs[u], gsem[u])
    def outer(t, c):
        for u in range(NB):
            j = t * NB + u
            pltpu.make_async_copy(y_hbm.at[row_v.at[j, 0]], bufs[u],
                                  gsem[u]).wait()
            pltpu.sync_copy(bufs[u], s_sh.at[col_v.at[j, 0]], add=True)
            @pl.when(j + NB < nb)
            def _():
                pltpu.async_copy(y_hbm.at[row_v.at[j + NB, 0]], bufs[u],
                                 gsem[u])
        return c
    lax.fori_loop(0, BPT // NB, outer, 0)
    @pl.when(sid < XTRA)
    def _():
        pltpu.make_async_copy(y_hbm.at[row_v.at[BPT, 0]], bufs[0],
                              gsem[0]).wait()
        pltpu.sync_copy(bufs[0], s_sh.at[col_v.at[BPT, 0]], add=True)
    plsc.subcore_barrier()
    pltpu.sync_copy(s_sh.at[pl.ds(sid * RPT, RPT)],
                    s_hbm.at[pl.ds(sid * RPT, RPT), pl.ds(cid * DH, DH)])


_gs_call = pl.kernel(
    _gs_body,
    out_type=jax.ShapeDtypeStruct((NPAD, D), jnp.float32),
    mesh=plsc.VectorSubcoreMesh(core_axis_name="c", subcore_axis_name="s",
                                num_cores=NC, num_subcores=NS),
    scratch_types=(
        [pltpu.VMEM((MAXB, 1, B), jnp.int32)] * 2
        + [pltpu.VMEM((B, DH), jnp.float32)] * NB
        + [pltpu.VMEM_SHARED((NPAD, DH), jnp.float32)]
        + [pltpu.SemaphoreType.DMA] * NB
    ),
    compiler_params=_UNTILED,
)


def _sigmoid(x):
    return 1.0 / (1.0 + jnp.exp(-x))


def _dis_col(dd_ref):
    dis = lax.rsqrt(dd_ref[...] + 1.0)
    return jnp.transpose(dis.reshape(1, RB))


def _y_body(x_ref, w_ref, wihT_ref, bi_ref, bh_ref, dd_ref, y_ref, wev_ref):
    @pl.when(pl.program_id(0) == 0)
    def _():
        g = jnp.dot(w_ref[...], wihT_ref[...],
                    preferred_element_type=jnp.float32)
        g = g + bi_ref[...] + bh_ref[...]
        i = g[:, :D]
        gg = g[:, 2 * D:3 * D]
        o = g[:, 3 * D:]
        c = _sigmoid(i) * jnp.tanh(gg)
        wev_ref[...] = _sigmoid(o) * jnp.tanh(c)
    y_ref[...] = jnp.dot(x_ref[...], wev_ref[...],
                         preferred_element_type=jnp.float32) * _dis_col(dd_ref)


def _y_call(X, W, W_ihT, bi, bh, dd):
    return pl.pallas_call(
        _y_body,
        grid=((N + RB - 1) // RB,),
        in_specs=[
            pl.BlockSpec((RB, D), lambda i: (i, 0)),
            pl.BlockSpec((D, D), lambda i: (0, 0)),
            pl.BlockSpec((D, 4 * D), lambda i: (0, 0)),
            pl.BlockSpec((1, 4 * D), lambda i: (0, 0)),
            pl.BlockSpec((1, 4 * D), lambda i: (0, 0)),
            pl.BlockSpec((RB,), lambda i: (i,)),
        ],
        out_specs=pl.BlockSpec((RB, D), lambda i: (i, 0)),
        out_shape=jax.ShapeDtypeStruct((N, D), jnp.float32),
        scratch_shapes=[pltpu.VMEM((D, D), jnp.float32)],
    )(X, W, W_ihT, bi, bh, dd)


def _out_body(s_ref, y_ref, dd_ref, o_ref):
    o_ref[...] = (s_ref[...] + y_ref[...]) * _dis_col(dd_ref)


def _out_call(s, y, dd):
    return pl.pallas_call(
        _out_body,
        grid=((N + RB - 1) // RB,),
        in_specs=[
            pl.BlockSpec((RB, D), lambda i: (i, 0)),
            pl.BlockSpec((RB, D), lambda i: (i, 0)),
            pl.BlockSpec((RB,), lambda i: (i,)),
        ],
        out_specs=pl.BlockSpec((RB, D), lambda i: (i, 0)),
        out_shape=jax.ShapeDtypeStruct((N, D), jnp.float32),
    )(s, y, dd)


def kernel(X, edge_index, W, W_ih, W_hh, b_ih, b_hh):
    ei3 = edge_index.reshape(2, NBT, B).transpose(1, 0, 2)

    d0, d1 = _deg_call(ei3)
    dd = d0 + d1
    Y = _y_call(X, W, W_ih.T, b_ih.reshape(1, 4 * D), b_hh.reshape(1, 4 * D),
                dd)
    y_view = Y.reshape(2 * N, DH)
    S = _gs_call(y_view, ei3)
    return _out_call(S, Y, dd)
